# baseline (speedup 1.0000x reference)
import dataclasses
import functools

import jax
import jax.numpy as jnp
from jax import lax
from jax.experimental import pallas as pl
from jax.experimental.pallas import tpu as pltpu
from jax.experimental.pallas import tpu_sc as plsc

_NUM_CORES = 2
_NUM_SUBCORES = 16
_NUM_WORKERS = _NUM_CORES * _NUM_SUBCORES
_CHUNK = 800
_CPY = 1000


def _params(tc_tiling):
    cp = pltpu.CompilerParams(use_tc_tiling_on_sc=tc_tiling)
    if "needs_layout_passes" in pltpu.CompilerParams.__dataclass_fields__:
        cp = dataclasses.replace(cp, needs_layout_passes=False)
    return cp


def _relayout(table2):
    rows = table2.shape[0]
    n_blocks = rows // _CPY
    mesh = plsc.VectorSubcoreMesh(core_axis_name="c", subcore_axis_name="s")

    @functools.partial(
        pl.kernel,
        mesh=mesh,
        out_type=jax.ShapeDtypeStruct(table2.shape, table2.dtype),
        scratch_types=[],
        compiler_params=_params(True),
    )
    def copy_kernel(in_hbm, out_hbm):
        wid = lax.axis_index("s") * _NUM_CORES + lax.axis_index("c")

        @pl.loop(0, (n_blocks + _NUM_WORKERS - 1) // _NUM_WORKERS)
        def _(k):
            b = k * _NUM_WORKERS + wid

            @pl.when(b < n_blocks)
            def _():
                sl = pl.ds(pl.multiple_of(b * _CPY, 8), _CPY)
                pltpu.sync_copy(in_hbm.at[sl], out_hbm.at[sl])

    return copy_kernel(table2)


def _gather_rows(idx_flat, table_lin, n, d):
    n_per_w = n // _NUM_WORKERS
    n_chunks = n_per_w // _CHUNK
    mesh = plsc.VectorSubcoreMesh(core_axis_name="c", subcore_axis_name="s")

    @functools.partial(
        pl.kernel,
        mesh=mesh,
        out_type=jax.ShapeDtypeStruct((n, d), jnp.float32),
        scratch_types=[
            pltpu.VMEM((_CHUNK,), jnp.int32),
            pltpu.VMEM((_CHUNK, d), jnp.float32),
            pltpu.SemaphoreType.DMA,
        ],
        compiler_params=_params(False),
    )
    def gather_kernel(table_hbm, idx_hbm, out_hbm, idx_v, rows_v, sem):
        wid = lax.axis_index("s") * _NUM_CORES + lax.axis_index("c")
        base = wid * n_per_w

        @pl.loop(0, n_chunks)
        def _(i):
            off = base + i * _CHUNK
            pltpu.sync_copy(idx_hbm.at[pl.ds(off, _CHUNK)], idx_v)
            pltpu.async_copy(table_hbm.at[idx_v], rows_v, sem).wait()
            pltpu.sync_copy(rows_v, out_hbm.at[pl.ds(off, _CHUNK)])

    return gather_kernel(table_lin, idx_flat)


def kernel(input_batch, table):
    b, l = input_batch.shape
    v, d = table.shape
    n = b * l
    idx_flat = input_batch.reshape(n)
    table2 = table.reshape(v // 2, 2 * d)
    table_lin = _relayout(table2).reshape(v, d)
    rows = _gather_rows(idx_flat, table_lin, n, d)
    embedded = rows.reshape(b, l, d)
    sequence_lengths = jnp.full((b,), float(l), dtype=jnp.float32)
    return (embedded, sequence_lengths)


# speedup vs baseline: 8.7151x; 8.7151x over previous
import dataclasses
import functools

import jax
import jax.numpy as jnp
from jax import lax
from jax.experimental import pallas as pl
from jax.experimental.pallas import tpu as pltpu
from jax.experimental.pallas import tpu_sc as plsc

_NUM_CORES = 2
_NUM_SUBCORES = 16
_NUM_WORKERS = _NUM_CORES * _NUM_SUBCORES
_CHUNK = 800
_CPY = 400


def _params(tc_tiling):
    cp = pltpu.CompilerParams(use_tc_tiling_on_sc=tc_tiling)
    if "needs_layout_passes" in pltpu.CompilerParams.__dataclass_fields__:
        cp = dataclasses.replace(cp, needs_layout_passes=False)
    return cp


def _relayout(table2):
    rows = table2.shape[0]
    n_blocks = rows // _CPY
    blocks_per_w = (n_blocks + _NUM_WORKERS - 1) // _NUM_WORKERS
    mesh = plsc.VectorSubcoreMesh(core_axis_name="c", subcore_axis_name="s")

    @functools.partial(
        pl.kernel,
        mesh=mesh,
        out_type=jax.ShapeDtypeStruct(table2.shape, table2.dtype),
        scratch_types=[
            pltpu.VMEM((2 * _CPY, 128), jnp.float32),
            pltpu.SemaphoreType.DMA,
            pltpu.SemaphoreType.DMA,
            pltpu.SemaphoreType.DMA,
            pltpu.SemaphoreType.DMA,
        ],
        compiler_params=_params(True),
    )
    def copy_kernel(in_hbm, out_hbm, buf, li0, li1, lo0, lo1):
        wid = lax.axis_index("s") * _NUM_CORES + lax.axis_index("c")
        lins = (li0, li1)
        louts = (lo0, lo1)

        def blk(k):
            b = k * _NUM_WORKERS + wid
            return b, pl.ds(pl.multiple_of(b * _CPY, 8), _CPY)

        def load(k):
            b, sl = blk(k)
            slot = k % 2

            @pl.when(b < n_blocks)
            def _():
                pltpu.async_copy(
                    in_hbm.at[sl], buf.at[pl.ds(slot * _CPY, _CPY)],
                    lins[slot])

        load(0)
        for k in range(blocks_per_w):
            slot = k % 2
            if k + 1 < blocks_per_w:
                load(k + 1)
            b, sl = blk(k)

            @pl.when(b < n_blocks)
            def _(slot=slot, sl=sl):
                pltpu.make_async_copy(
                    in_hbm.at[sl], buf.at[pl.ds(slot * _CPY, _CPY)],
                    lins[slot]).wait()
                pltpu.async_copy(
                    buf.at[pl.ds(slot * _CPY, _CPY)], out_hbm.at[sl],
                    louts[slot]).wait()

    return copy_kernel(table2)


def _gather_rows(idx_flat, table_lin, n, d):
    n_per_w = n // _NUM_WORKERS
    n_chunks = n_per_w // _CHUNK
    mesh = plsc.VectorSubcoreMesh(core_axis_name="c", subcore_axis_name="s")

    @functools.partial(
        pl.kernel,
        mesh=mesh,
        out_type=jax.ShapeDtypeStruct((n, d), jnp.float32),
        scratch_types=[
            pltpu.VMEM((_CHUNK,), jnp.int32),
            pltpu.VMEM((_CHUNK, d), jnp.float32),
            pltpu.SemaphoreType.DMA,
        ],
        compiler_params=_params(False),
    )
    def gather_kernel(table_hbm, idx_hbm, out_hbm, idx_v, rows_v, sem):
        wid = lax.axis_index("s") * _NUM_CORES + lax.axis_index("c")
        base = wid * n_per_w

        @pl.loop(0, n_chunks)
        def _(i):
            off = base + i * _CHUNK
            pltpu.sync_copy(idx_hbm.at[pl.ds(off, _CHUNK)], idx_v)
            pltpu.async_copy(table_hbm.at[idx_v], rows_v, sem).wait()
            pltpu.sync_copy(rows_v, out_hbm.at[pl.ds(off, _CHUNK)])

    return gather_kernel(table_lin, idx_flat)


def kernel(input_batch, table):
    b, l = input_batch.shape
    v, d = table.shape
    n = b * l
    idx_flat = input_batch.reshape(n)
    table2 = table.reshape(v // 2, 2 * d)
    table_lin = _relayout(table2).reshape(v, d)
    rows = _gather_rows(idx_flat, table_lin, n, d)
    embedded = rows.reshape(b, l, d)
    sequence_lengths = jnp.full((b,), float(l), dtype=jnp.float32)
    return (embedded, sequence_lengths)


# speedup vs baseline: 10.8309x; 1.2428x over previous
import functools

import jax
import jax.numpy as jnp
from jax import lax
from jax.experimental import pallas as pl
from jax.experimental.pallas import tpu as pltpu
from jax.experimental.pallas import tpu_sc as plsc

_NUM_CORES = 2
_NUM_SUBCORES = 16
_NUM_WORKERS = _NUM_CORES * _NUM_SUBCORES
_CHUNK = 800


def _gather_rows(idx_flat, table, n, d):
    n_per_w = n // _NUM_WORKERS
    n_chunks = n_per_w // _CHUNK
    mesh = plsc.VectorSubcoreMesh(core_axis_name="c", subcore_axis_name="s")

    @functools.partial(
        pl.kernel,
        mesh=mesh,
        out_type=jax.ShapeDtypeStruct((n, d), jnp.float32),
        scratch_types=[
            pltpu.VMEM((_CHUNK,), jnp.int32),
            pltpu.VMEM((_CHUNK, d), jnp.float32),
            pltpu.SemaphoreType.DMA,
        ],
        compiler_params=pltpu.CompilerParams(use_tc_tiling_on_sc=False),
    )
    def gather_kernel(table_hbm, idx_hbm, out_hbm, idx_v, rows_v, sem):
        wid = lax.axis_index("s") * _NUM_CORES + lax.axis_index("c")
        base = wid * n_per_w

        @pl.loop(0, n_chunks)
        def _(i):
            off = base + i * _CHUNK
            pltpu.sync_copy(idx_hbm.at[pl.ds(off, _CHUNK)], idx_v)
            pltpu.async_copy(table_hbm.at[idx_v], rows_v, sem).wait()
            pltpu.sync_copy(rows_v, out_hbm.at[pl.ds(off, _CHUNK)])

    return gather_kernel(table, idx_flat)


def kernel(input_batch, table):
    b, l = input_batch.shape
    v, d = table.shape
    n = b * l
    idx_flat = input_batch.reshape(n)
    rows = _gather_rows(idx_flat, table, n, d)
    embedded = rows.reshape(b, l, d)
    sequence_lengths = jnp.full((b,), float(l), dtype=jnp.float32)
    return (embedded, sequence_lengths)
